# baseline (speedup 1.0000x reference)
import jax
import jax.numpy as jnp
from jax.experimental import pallas as pl

DIM = 192
NUM_HEADS = 6
WS = 16
N = WS * WS
TOPK = 128
HEAD_DIM = DIM // NUM_HEADS
SCALE = HEAD_DIM ** -0.5
EPS = 1e-20
TABLE = (2 * WS - 1) * (2 * WS - 1)


def _proj_body(x_ref, w_ref, b_ref, o_ref):
    o_ref[0] = (
        jax.lax.dot_general(
            x_ref[0], w_ref[...], (((1,), (1,)), ((), ())),
            preferred_element_type=jnp.float32,
        )
        + b_ref[...]
    )


def kernel(qkvp, rpb_table, proj_w, proj_b, rpi):
    b_, n, c4 = qkvp.shape
    c = c4 // 4
    hd = c // NUM_HEADS
    qkvp_r = jnp.transpose(qkvp.reshape(b_, n, 4, NUM_HEADS, hd), (2, 0, 3, 1, 4))
    q, k, v, v_lepe = qkvp_r[0], qkvp_r[1], qkvp_r[2], qkvp_r[3]
    q = q * SCALE
    attn = jnp.einsum('bhnd,bhmd->bhnm', q, k)
    rpb = rpb_table[rpi.reshape(-1)].reshape(n, n, -1)
    rpb = jnp.transpose(rpb, (2, 0, 1))[None]
    attn = attn + rpb
    attn = jax.nn.softmax(attn, axis=-1)
    topk_values, topk_indices = jax.lax.top_k(attn, TOPK)
    attn_sp = topk_values / (jnp.sum(topk_values, axis=-1, keepdims=True) + EPS)
    bh = b_ * NUM_HEADS
    idx = topk_indices.reshape(bh * n, TOPK)
    vals = attn_sp.reshape(bh * n, TOPK)
    rows = jnp.arange(bh * n)[:, None]
    full = jnp.zeros((bh * n, n), dtype=attn_sp.dtype).at[rows, idx].add(vals)
    vv = v.reshape(bh, n, hd)
    out = jnp.einsum('bnm,bmc->bnc', full.reshape(bh, n, n), vv).reshape(b_, NUM_HEADS, n, hd)
    xin = jnp.transpose(out + v_lepe, (0, 2, 1, 3)).reshape(b_, n, c)

    x = pl.pallas_call(
        _proj_body,
        grid=(b_,),
        in_specs=[
            pl.BlockSpec((1, n, c), lambda i: (i, 0, 0)),
            pl.BlockSpec((c, c), lambda i: (0, 0)),
            pl.BlockSpec((1, c), lambda i: (0, 0)),
        ],
        out_specs=pl.BlockSpec((1, n, c), lambda i: (i, 0, 0)),
        out_shape=jax.ShapeDtypeStruct((b_, n, c), jnp.float32),
    )(xin, proj_w, proj_b.reshape(1, c))
    return x, attn_sp, topk_indices


# speedup vs baseline: 16.7662x; 16.7662x over previous
import functools

import jax
import jax.numpy as jnp
from jax.experimental import pallas as pl
from jax.experimental.pallas import tpu as pltpu

DIM = 192
NUM_HEADS = 6
WS = 16
N = WS * WS
TOPK = 128
HEAD_DIM = DIM // NUM_HEADS
SCALE = HEAD_DIM ** -0.5
EPS = 1e-20
TABLE = (2 * WS - 1) * (2 * WS - 1)


def _attn_body(qkvp_ref, rpb_ref, pw_ref, pb_ref, x_ref, pv_ref, pi_ref, xacc):
    h = pl.program_id(1)
    q = qkvp_ref[0, 0, 0]
    k = qkvp_ref[0, 1, 0]
    v = qkvp_ref[0, 2, 0]
    lepe = qkvp_ref[0, 3, 0]

    at = jax.lax.dot_general(
        k, q * SCALE, (((1,), (1,)), ((), ())), preferred_element_type=jnp.float32
    ) + rpb_ref[h]

    val = at
    idx = jax.lax.broadcasted_iota(jnp.int32, (N, N), 0)
    pcol = jax.lax.broadcasted_iota(jnp.int32, (N, 1), 0)
    for kk in range(8):
        desc = ((pcol >> (kk + 1)) & 1) == 0
        for j in range(kk, -1, -1):
            d = 1 << j
            up = ((pcol >> j) & 1) == 1
            a_msk = jnp.logical_xor(desc, up)
            vp = pltpu.roll(val, N - d, 0)
            vm = pltpu.roll(val, d, 0)
            partner_v = jnp.where(up, vm, vp)
            ip = pltpu.roll(idx, N - d, 0)
            im = pltpu.roll(idx, d, 0)
            partner_i = jnp.where(up, im, ip)
            sp = (partner_v > val) == a_msk
            val = jnp.where(sp, partner_v, val)
            idx = jnp.where(sp, partner_i, idx)

    m = jnp.max(val[0:1], axis=0, keepdims=True)
    es = jnp.exp(val[:TOPK] - m)
    is_ = idx[:TOPK]
    z = jnp.sum(es, axis=0, keepdims=True)
    rz = 1.0 / (z + EPS)
    pv_ref[0, 0] = jnp.transpose(es * rz)
    pi_ref[0, 0] = jnp.transpose(is_)

    t128 = val[TOPK - 1:TOPK]
    e = jnp.exp(at - m)
    pm = jnp.where(at >= t128, e, 0.0) * rz
    out_h = jax.lax.dot_general(
        pm, v, (((0,), (0,)), ((), ())), preferred_element_type=jnp.float32
    ) + lepe

    partial = jax.lax.dot_general(
        out_h, pw_ref[0], (((1,), (1,)), ((), ())),
        preferred_element_type=jnp.float32,
    )

    @pl.when(h == 0)
    def _():
        xacc[...] = partial

    @pl.when(h > 0)
    def _():
        xacc[...] += partial

    @pl.when(h == NUM_HEADS - 1)
    def _():
        x_ref[0] = xacc[...] + pb_ref[...]


def kernel(qkvp, rpb_table, proj_w, proj_b, rpi):
    b_, n, c4 = qkvp.shape
    qkvp_r = jnp.transpose(
        qkvp.reshape(b_, n, 4, NUM_HEADS, HEAD_DIM), (0, 2, 3, 1, 4)
    )
    rpb_t = jnp.transpose(rpb_table[rpi], (2, 1, 0))
    pw_r = jnp.transpose(proj_w.reshape(DIM, NUM_HEADS, HEAD_DIM), (1, 0, 2))

    grid = (b_, NUM_HEADS)
    x, pfa_vals, pfa_idx = pl.pallas_call(
        _attn_body,
        grid=grid,
        in_specs=[
            pl.BlockSpec((1, 4, 1, n, HEAD_DIM), lambda w, h: (w, 0, h, 0, 0)),
            pl.BlockSpec((NUM_HEADS, n, n), lambda w, h: (0, 0, 0)),
            pl.BlockSpec((1, DIM, HEAD_DIM), lambda w, h: (h, 0, 0)),
            pl.BlockSpec((1, DIM), lambda w, h: (0, 0)),
        ],
        out_specs=[
            pl.BlockSpec((1, n, DIM), lambda w, h: (w, 0, 0)),
            pl.BlockSpec((1, 1, n, TOPK), lambda w, h: (w, h, 0, 0)),
            pl.BlockSpec((1, 1, n, TOPK), lambda w, h: (w, h, 0, 0)),
        ],
        out_shape=[
            jax.ShapeDtypeStruct((b_, n, DIM), jnp.float32),
            jax.ShapeDtypeStruct((b_, NUM_HEADS, n, TOPK), jnp.float32),
            jax.ShapeDtypeStruct((b_, NUM_HEADS, n, TOPK), jnp.int32),
        ],
        scratch_shapes=[pltpu.VMEM((n, DIM), jnp.float32)],
        compiler_params=pltpu.CompilerParams(
            dimension_semantics=("parallel", "arbitrary"),
        ),
    )(qkvp_r, rpb_t, pw_r, proj_b.reshape(1, DIM))
    return x, pfa_vals, pfa_idx


# speedup vs baseline: 36.9762x; 2.2054x over previous
import functools

import jax
import jax.numpy as jnp
from jax.experimental import pallas as pl
from jax.experimental.pallas import tpu as pltpu

DIM = 192
NUM_HEADS = 6
WS = 16
N = WS * WS
TOPK = 128
HEAD_DIM = DIM // NUM_HEADS
SCALE = HEAD_DIM ** -0.5
EPS = 1e-20
TABLE = (2 * WS - 1) * (2 * WS - 1)


def _attn_body(q_ref, k_ref, v_ref, l_ref, rpb_ref, pw_ref, pb_ref,
               x_ref, pv_ref, pi_ref, xacc):
    h = pl.program_id(1)
    q = q_ref[0, 0]
    k = k_ref[0, 0]
    v = v_ref[0, 0]
    lepe = l_ref[0, 0]

    at = jax.lax.dot_general(
        k, q * SCALE, (((1,), (1,)), ((), ())), preferred_element_type=jnp.float32
    ) + rpb_ref[h]

    iota8 = jax.lax.broadcasted_iota(jnp.int32, (8, 1), 0)
    up8 = [((iota8 >> j) & 1) == 1 for j in range(3)]
    vch = [at[c * 8:(c + 1) * 8] for c in range(32)]
    ich = [jax.lax.broadcasted_iota(jnp.int32, (8, N), 0) + (c * 8)
           for c in range(32)]

    def cmpx_chunks(lo_v, hi_v, lo_i, hi_i, desc):
        mx = jnp.maximum(lo_v, hi_v)
        mn = jnp.minimum(lo_v, hi_v)
        mx_i = jnp.where(mx == hi_v, hi_i, lo_i)
        mn_i = jnp.where(mn == lo_v, lo_i, hi_i)
        if desc:
            return mx, mn, mx_i, mn_i
        return mn, mx, mn_i, mx_i

    for kk in range(8):
        for j in range(kk, -1, -1):
            d = 1 << j
            if d >= 8:
                cd = d // 8
                for base in range(0, 32, 2 * cd):
                    if kk == 7 and j < 7 and base >= 16:
                        continue
                    desc = (((base * 8) >> (kk + 1)) & 1) == 0
                    for o in range(cd):
                        a, b = base + o, base + o + cd
                        vch[a], vch[b], ich[a], ich[b] = cmpx_chunks(
                            vch[a], vch[b], ich[a], ich[b], desc)
            else:
                up = up8[j]
                for c in range(16 if kk == 7 else 32):
                    if kk >= 2:
                        desc = (((c * 8) >> (kk + 1)) & 1) == 0
                        a_msk = jnp.logical_not(up) if desc else up
                    else:
                        desc8 = ((iota8 >> (kk + 1)) & 1) == 0
                        a_msk = desc8 != up
                    cv = vch[c]
                    ci = ich[c]
                    if d == 4:
                        pv = pltpu.roll(cv, 4, 0)
                        pi = pltpu.roll(ci, 4, 0)
                    else:
                        pv = jnp.where(up, pltpu.roll(cv, d, 0),
                                       pltpu.roll(cv, 8 - d, 0))
                        pi = jnp.where(up, pltpu.roll(ci, d, 0),
                                       pltpu.roll(ci, 8 - d, 0))
                    sp = (pv > cv) == a_msk
                    vch[c] = jnp.where(sp, pv, cv)
                    ich[c] = jnp.where(sp, pi, ci)

    m = vch[0][0:1]
    vs = jnp.concatenate(vch[:TOPK // 8], axis=0)
    is_ = jnp.concatenate(ich[:TOPK // 8], axis=0)
    es = jnp.exp(vs - m)
    z = jnp.sum(es, axis=0, keepdims=True)
    rz = 1.0 / (z + EPS)
    pv_ref[0, 0] = jnp.transpose(es * rz)
    pi_ref[0, 0] = jnp.transpose(is_)

    t128 = vch[TOPK // 8 - 1][7:8]
    e = jnp.exp(at - m)
    pm = jnp.where(at >= t128, e, 0.0) * rz
    out_h = jax.lax.dot_general(
        pm, v, (((0,), (0,)), ((), ())), preferred_element_type=jnp.float32
    ) + lepe

    partial = jax.lax.dot_general(
        out_h, pw_ref[0], (((1,), (1,)), ((), ())),
        preferred_element_type=jnp.float32,
    )

    @pl.when(h == 0)
    def _():
        xacc[...] = partial

    @pl.when(h > 0)
    def _():
        xacc[...] += partial

    @pl.when(h == NUM_HEADS - 1)
    def _():
        x_ref[0] = xacc[...] + pb_ref[...]


def kernel(qkvp, rpb_table, proj_w, proj_b, rpi):
    b_, n, c4 = qkvp.shape
    qkvp_r = jnp.transpose(
        qkvp.reshape(b_, n, 4 * NUM_HEADS, HEAD_DIM), (2, 0, 1, 3))
    rpb_t = jnp.transpose(rpb_table[rpi], (2, 1, 0))
    pw_r = jnp.transpose(proj_w.reshape(DIM, NUM_HEADS, HEAD_DIM), (1, 0, 2))

    grid = (b_, NUM_HEADS)
    x, pfa_vals, pfa_idx = pl.pallas_call(
        _attn_body,
        grid=grid,
        in_specs=[
            pl.BlockSpec((1, 1, n, HEAD_DIM), lambda w, h: (h, w, 0, 0)),
            pl.BlockSpec((1, 1, n, HEAD_DIM),
                         lambda w, h: (NUM_HEADS + h, w, 0, 0)),
            pl.BlockSpec((1, 1, n, HEAD_DIM),
                         lambda w, h: (2 * NUM_HEADS + h, w, 0, 0)),
            pl.BlockSpec((1, 1, n, HEAD_DIM),
                         lambda w, h: (3 * NUM_HEADS + h, w, 0, 0)),
            pl.BlockSpec((NUM_HEADS, n, n), lambda w, h: (0, 0, 0)),
            pl.BlockSpec((1, DIM, HEAD_DIM), lambda w, h: (h, 0, 0)),
            pl.BlockSpec((1, DIM), lambda w, h: (0, 0)),
        ],
        out_specs=[
            pl.BlockSpec((1, n, DIM), lambda w, h: (w, 0, 0)),
            pl.BlockSpec((1, 1, n, TOPK), lambda w, h: (w, h, 0, 0)),
            pl.BlockSpec((1, 1, n, TOPK), lambda w, h: (w, h, 0, 0)),
        ],
        out_shape=[
            jax.ShapeDtypeStruct((b_, n, DIM), jnp.float32),
            jax.ShapeDtypeStruct((b_, NUM_HEADS, n, TOPK), jnp.float32),
            jax.ShapeDtypeStruct((b_, NUM_HEADS, n, TOPK), jnp.int32),
        ],
        scratch_shapes=[pltpu.VMEM((n, DIM), jnp.float32)],
        compiler_params=pltpu.CompilerParams(
            dimension_semantics=("parallel", "arbitrary"),
        ),
    )(qkvp_r, qkvp_r, qkvp_r, qkvp_r, rpb_t, pw_r, proj_b.reshape(1, DIM))
    return x, pfa_vals, pfa_idx


# speedup vs baseline: 37.0708x; 1.0026x over previous
import functools

import jax
import jax.numpy as jnp
from jax.experimental import pallas as pl
from jax.experimental.pallas import tpu as pltpu

DIM = 192
NUM_HEADS = 6
WS = 16
N = WS * WS
TOPK = 128
HEAD_DIM = DIM // NUM_HEADS
SCALE = HEAD_DIM ** -0.5
EPS = 1e-20
TABLE = (2 * WS - 1) * (2 * WS - 1)


def _attn_body(q_ref, k_ref, v_ref, l_ref, rpb_ref, pw_ref, pb_ref,
               x_ref, pv_ref, pi_ref, xacc):
    h = pl.program_id(1)
    q = q_ref[0, 0]
    k = k_ref[0, 0]
    v = v_ref[0, 0]
    lepe = l_ref[0, 0]

    at = jax.lax.dot_general(
        k, q * SCALE, (((1,), (1,)), ((), ())), preferred_element_type=jnp.float32
    ) + rpb_ref[h]

    iota8 = jax.lax.broadcasted_iota(jnp.int32, (8, 1), 0)
    up8 = [((iota8 >> j) & 1) == 1 for j in range(3)]
    vch = [at[c * 8:(c + 1) * 8] for c in range(32)]
    ich = [jax.lax.broadcasted_iota(jnp.int32, (8, N), 0) + (c * 8)
           for c in range(32)]

    def cmpx_chunks(lo_v, hi_v, lo_i, hi_i, desc):
        p = (hi_v > lo_v) if desc else (lo_v > hi_v)
        return (jnp.where(p, hi_v, lo_v), jnp.where(p, lo_v, hi_v),
                jnp.where(p, hi_i, lo_i), jnp.where(p, lo_i, hi_i))

    for kk in range(8):
        for j in range(kk, -1, -1):
            d = 1 << j
            if d >= 8:
                cd = d // 8
                for base in range(0, 32, 2 * cd):
                    if kk == 7 and j < 7 and base >= 16:
                        continue
                    desc = (((base * 8) >> (kk + 1)) & 1) == 0
                    for o in range(cd):
                        a, b = base + o, base + o + cd
                        if kk == 7 and j == 7:
                            p = vch[b] > vch[a]
                            vch[a] = jnp.where(p, vch[b], vch[a])
                            ich[a] = jnp.where(p, ich[b], ich[a])
                        else:
                            vch[a], vch[b], ich[a], ich[b] = cmpx_chunks(
                                vch[a], vch[b], ich[a], ich[b], desc)
            else:
                up = up8[j]
                for c in range(16 if kk == 7 else 32):
                    if kk >= 2:
                        desc = (((c * 8) >> (kk + 1)) & 1) == 0
                        a_msk = jnp.logical_not(up) if desc else up
                    else:
                        desc8 = ((iota8 >> (kk + 1)) & 1) == 0
                        a_msk = desc8 != up
                    cv = vch[c]
                    ci = ich[c]
                    if d == 4:
                        pv = pltpu.roll(cv, 4, 0)
                        pi = pltpu.roll(ci, 4, 0)
                    else:
                        pv = jnp.where(up, pltpu.roll(cv, d, 0),
                                       pltpu.roll(cv, 8 - d, 0))
                        pi = jnp.where(up, pltpu.roll(ci, d, 0),
                                       pltpu.roll(ci, 8 - d, 0))
                    sp = (pv > cv) == a_msk
                    vch[c] = jnp.where(sp, pv, cv)
                    ich[c] = jnp.where(sp, pi, ci)

    m = vch[0][0:1]
    vs = jnp.concatenate(vch[:TOPK // 8], axis=0)
    is_ = jnp.concatenate(ich[:TOPK // 8], axis=0)
    es = jnp.exp(vs - m)
    z = jnp.sum(es, axis=0, keepdims=True)
    rz = 1.0 / (z + EPS)
    pv_ref[0, 0] = jnp.transpose(es * rz)
    pi_ref[0, 0] = jnp.transpose(is_)

    t128 = vch[TOPK // 8 - 1][7:8]
    e = jnp.exp(at - m)
    pm = jnp.where(at >= t128, e, 0.0) * rz
    out_h = jax.lax.dot_general(
        pm, v, (((0,), (0,)), ((), ())), preferred_element_type=jnp.float32
    ) + lepe

    partial = jax.lax.dot_general(
        out_h, pw_ref[0], (((1,), (1,)), ((), ())),
        preferred_element_type=jnp.float32,
    )

    @pl.when(h == 0)
    def _():
        xacc[...] = partial

    @pl.when(h > 0)
    def _():
        xacc[...] += partial

    @pl.when(h == NUM_HEADS - 1)
    def _():
        x_ref[0] = xacc[...] + pb_ref[...]


def kernel(qkvp, rpb_table, proj_w, proj_b, rpi):
    b_, n, c4 = qkvp.shape
    qkvp_r = jnp.transpose(
        qkvp.reshape(b_, n, 4 * NUM_HEADS, HEAD_DIM), (2, 0, 1, 3))
    rpb_t = jnp.transpose(rpb_table[rpi], (2, 1, 0))
    pw_r = jnp.transpose(proj_w.reshape(DIM, NUM_HEADS, HEAD_DIM), (1, 0, 2))

    grid = (b_, NUM_HEADS)
    x, pfa_vals, pfa_idx = pl.pallas_call(
        _attn_body,
        grid=grid,
        in_specs=[
            pl.BlockSpec((1, 1, n, HEAD_DIM), lambda w, h: (h, w, 0, 0)),
            pl.BlockSpec((1, 1, n, HEAD_DIM),
                         lambda w, h: (NUM_HEADS + h, w, 0, 0)),
            pl.BlockSpec((1, 1, n, HEAD_DIM),
                         lambda w, h: (2 * NUM_HEADS + h, w, 0, 0)),
            pl.BlockSpec((1, 1, n, HEAD_DIM),
                         lambda w, h: (3 * NUM_HEADS + h, w, 0, 0)),
            pl.BlockSpec((NUM_HEADS, n, n), lambda w, h: (0, 0, 0)),
            pl.BlockSpec((1, DIM, HEAD_DIM), lambda w, h: (h, 0, 0)),
            pl.BlockSpec((1, DIM), lambda w, h: (0, 0)),
        ],
        out_specs=[
            pl.BlockSpec((1, n, DIM), lambda w, h: (w, 0, 0)),
            pl.BlockSpec((1, 1, n, TOPK), lambda w, h: (w, h, 0, 0)),
            pl.BlockSpec((1, 1, n, TOPK), lambda w, h: (w, h, 0, 0)),
        ],
        out_shape=[
            jax.ShapeDtypeStruct((b_, n, DIM), jnp.float32),
            jax.ShapeDtypeStruct((b_, NUM_HEADS, n, TOPK), jnp.float32),
            jax.ShapeDtypeStruct((b_, NUM_HEADS, n, TOPK), jnp.int32),
        ],
        scratch_shapes=[pltpu.VMEM((n, DIM), jnp.float32)],
        compiler_params=pltpu.CompilerParams(
            dimension_semantics=("parallel", "arbitrary"),
        ),
    )(qkvp_r, qkvp_r, qkvp_r, qkvp_r, rpb_t, pw_r, proj_b.reshape(1, DIM))
    return x, pfa_vals, pfa_idx


# speedup vs baseline: 44.3113x; 1.1953x over previous
import jax
import jax.numpy as jnp
from jax.experimental import pallas as pl
from jax.experimental.pallas import tpu as pltpu

DIM = 192
NUM_HEADS = 6
WS = 16
N = WS * WS
TOPK = 128
HEAD_DIM = DIM // NUM_HEADS
SCALE = HEAD_DIM ** -0.5
EPS = 1e-20
TABLE = (2 * WS - 1) * (2 * WS - 1)


def _attn_body(qkvp_ref, rpb_ref, pw_ref, pb_ref, x_ref, pv_ref, pi_ref):
    allr = qkvp_ref[0]
    iota8 = jax.lax.broadcasted_iota(jnp.int32, (8, 1), 0)
    up8 = [((iota8 >> j) & 1) == 1 for j in range(3)]
    iotas = [jax.lax.broadcasted_iota(jnp.int32, (8, N), 0) + (c * 8)
             for c in range(32)]

    def cmpx_chunks(lo_v, hi_v, lo_i, hi_i, desc):
        p = (hi_v > lo_v) if desc else (lo_v > hi_v)
        return (jnp.where(p, hi_v, lo_v), jnp.where(p, lo_v, hi_v),
                jnp.where(p, hi_i, lo_i), jnp.where(p, lo_i, hi_i))

    acc = None
    for h in range(NUM_HEADS):
        q = allr[:, h * HEAD_DIM:(h + 1) * HEAD_DIM]
        k = allr[:, (NUM_HEADS + h) * HEAD_DIM:(NUM_HEADS + h + 1) * HEAD_DIM]
        v = allr[:, (2 * NUM_HEADS + h) * HEAD_DIM:
                 (2 * NUM_HEADS + h + 1) * HEAD_DIM]
        lepe = allr[:, (3 * NUM_HEADS + h) * HEAD_DIM:
                    (3 * NUM_HEADS + h + 1) * HEAD_DIM]

        at = jax.lax.dot_general(
            k, q * SCALE, (((1,), (1,)), ((), ())),
            preferred_element_type=jnp.float32,
        ) + rpb_ref[h]

        vch = [at[c * 8:(c + 1) * 8] for c in range(32)]
        ich = list(iotas)

        for kk in range(8):
            for j in range(kk, -1, -1):
                d = 1 << j
                if d >= 8:
                    cd = d // 8
                    for base in range(0, 32, 2 * cd):
                        if kk == 7 and j < 7 and base >= 16:
                            continue
                        desc = (((base * 8) >> (kk + 1)) & 1) == 0
                        for o in range(cd):
                            a, b = base + o, base + o + cd
                            if kk == 7 and j == 7:
                                p = vch[b] > vch[a]
                                vch[a] = jnp.where(p, vch[b], vch[a])
                                ich[a] = jnp.where(p, ich[b], ich[a])
                            else:
                                vch[a], vch[b], ich[a], ich[b] = cmpx_chunks(
                                    vch[a], vch[b], ich[a], ich[b], desc)
                else:
                    up = up8[j]
                    for c in range(16 if kk == 7 else 32):
                        if kk >= 2:
                            desc = (((c * 8) >> (kk + 1)) & 1) == 0
                            a_msk = jnp.logical_not(up) if desc else up
                        else:
                            desc8 = ((iota8 >> (kk + 1)) & 1) == 0
                            a_msk = desc8 != up
                        cv = vch[c]
                        ci = ich[c]
                        if d == 4:
                            pv = pltpu.roll(cv, 4, 0)
                            pi = pltpu.roll(ci, 4, 0)
                        else:
                            pv = jnp.where(up, pltpu.roll(cv, d, 0),
                                           pltpu.roll(cv, 8 - d, 0))
                            pi = jnp.where(up, pltpu.roll(ci, d, 0),
                                           pltpu.roll(ci, 8 - d, 0))
                        sp = (pv > cv) == a_msk
                        vch[c] = jnp.where(sp, pv, cv)
                        ich[c] = jnp.where(sp, pi, ci)

        m = vch[0][0:1]
        vs = jnp.concatenate(vch[:TOPK // 8], axis=0)
        is_ = jnp.concatenate(ich[:TOPK // 8], axis=0)
        es = jnp.exp(vs - m)
        z = jnp.sum(es, axis=0, keepdims=True)
        rz = 1.0 / (z + EPS)
        pv_ref[0, h] = jnp.transpose(es * rz)
        pi_ref[0, h] = jnp.transpose(is_)

        t128 = vch[TOPK // 8 - 1][7:8]
        e = jnp.exp(at - m)
        pm = jnp.where(at >= t128, e, 0.0) * rz
        out_h = jax.lax.dot_general(
            pm, v, (((0,), (0,)), ((), ())), preferred_element_type=jnp.float32
        ) + lepe

        partial = jax.lax.dot_general(
            out_h, pw_ref[h], (((1,), (1,)), ((), ())),
            preferred_element_type=jnp.float32,
        )
        acc = partial if acc is None else acc + partial

    x_ref[0] = acc + pb_ref[...]


def kernel(qkvp, rpb_table, proj_w, proj_b, rpi):
    b_, n, c4 = qkvp.shape
    rpb_t = jnp.transpose(rpb_table[rpi], (2, 1, 0))
    pw_r = jnp.transpose(proj_w.reshape(DIM, NUM_HEADS, HEAD_DIM), (1, 0, 2))

    grid = (b_,)
    x, pfa_vals, pfa_idx = pl.pallas_call(
        _attn_body,
        grid=grid,
        in_specs=[
            pl.BlockSpec((1, n, c4), lambda w: (w, 0, 0)),
            pl.BlockSpec((NUM_HEADS, n, n), lambda w: (0, 0, 0)),
            pl.BlockSpec((NUM_HEADS, DIM, HEAD_DIM), lambda w: (0, 0, 0)),
            pl.BlockSpec((1, DIM), lambda w: (0, 0)),
        ],
        out_specs=[
            pl.BlockSpec((1, n, DIM), lambda w: (w, 0, 0)),
            pl.BlockSpec((1, NUM_HEADS, n, TOPK), lambda w: (w, 0, 0, 0)),
            pl.BlockSpec((1, NUM_HEADS, n, TOPK), lambda w: (w, 0, 0, 0)),
        ],
        out_shape=[
            jax.ShapeDtypeStruct((b_, n, DIM), jnp.float32),
            jax.ShapeDtypeStruct((b_, NUM_HEADS, n, TOPK), jnp.float32),
            jax.ShapeDtypeStruct((b_, NUM_HEADS, n, TOPK), jnp.int32),
        ],
        compiler_params=pltpu.CompilerParams(
            dimension_semantics=("parallel",),
        ),
    )(qkvp, rpb_t, pw_r, proj_b.reshape(1, DIM))
    return x, pfa_vals, pfa_idx
